# baseline (speedup 1.0000x reference)
import functools
import math

import jax
import jax.numpy as jnp
from jax import lax
from jax.experimental import pallas as pl
from jax.experimental.pallas import tpu as pltpu
from jax.experimental.pallas import tpu_sc as plsc

_VOCAB = 1000000
_D = 32
_BATCH = 4096
_SEQ = 200
_SCALE = math.sqrt(_D)

_NC = 2
_NS = 16
_NW = _NC * _NS

_N = _BATCH * _SEQ
_ROWS_PW = _BATCH // _NW
_RPB = 4
_C = _RPB * _SEQ
_NCHUNK = _ROWS_PW // _RPB
_UNROLL = 8

_mesh = plsc.VectorSubcoreMesh(core_axis_name="c", subcore_axis_name="s")


@functools.partial(
    pl.kernel,
    mesh=_mesh,
    out_type=jax.ShapeDtypeStruct((_BATCH, _SEQ, _D), jnp.float32),
    scratch_types=[
        pltpu.VMEM((_C,), jnp.int32),
        pltpu.VMEM((_C, _D), jnp.float32),
        pltpu.SemaphoreType.DMA,
        pltpu.SemaphoreType.DMA,
    ],
    compiler_params=pltpu.CompilerParams(use_tc_tiling_on_sc=False),
)
def _emb_lookup(x_hbm, tab_hbm, out_hbm, idx_v, rows_v, gsem, ssem):
    wid = lax.axis_index("s") * _NC + lax.axis_index("c")
    row0 = wid * _ROWS_PW

    def chunk_body(ci, carry):
        r0 = row0 + ci * _RPB
        pltpu.sync_copy(x_hbm.at[pl.ds(r0 * _SEQ, _C)], idx_v)
        pltpu.async_copy(tab_hbm.at[idx_v], rows_v, gsem).wait()

        def scale_body(i, carry2):
            k0 = i * _UNROLL
            for u in range(_UNROLL):
                for h in range(_D // 16):
                    sl = (k0 + u, pl.ds(h * 16, 16))
                    rows_v[sl] = rows_v[sl] * _SCALE
            return carry2

        lax.fori_loop(0, _C // _UNROLL, scale_body, 0)

        for j in range(_RPB):
            pltpu.async_copy(
                rows_v.at[pl.ds(j * _SEQ, _SEQ)], out_hbm.at[r0 + j], ssem
            )
        for j in range(_RPB):
            pltpu.make_async_copy(
                rows_v.at[pl.ds(j * _SEQ, _SEQ)], out_hbm.at[r0 + j], ssem
            ).wait()
        return carry

    lax.fori_loop(0, _NCHUNK, chunk_body, 0)


def kernel(x, emb_table):
    return _emb_lookup(x.reshape(_N), emb_table)
